# baseline (speedup 1.0000x reference)
import functools

import jax
import jax.numpy as jnp
from jax import lax
from jax.experimental import pallas as pl
from jax.experimental.pallas import tpu as pltpu
from jax.experimental.pallas import tpu_sc as plsc

_NUM_ROWS = 424
_PAIR_DIM = 128
_LAG_DIM = 16
_OUT_DIM = _PAIR_DIM + _LAG_DIM
_NUM_LAGS = 4
_CHUNK = 32
_GROUPS = _CHUNK // 16
_ACTIVE = (_NUM_ROWS + _CHUNK - 1) // _CHUNK
_NUM_CORES = 1


def _take(v, i):
    dnums = lax.GatherDimensionNumbers(
        offset_dims=(), collapsed_slice_dims=(0,), start_index_map=(0,))
    return lax.gather(v, i[:, None], dnums, slice_sizes=(1,),
                      mode=lax.GatherScatterMode.PROMISE_IN_BOUNDS)


def _body(idxp_hbm, idxl_hbm, pair_hbm, lag_hbm, out_hbm,
          idxp_v, idxl_v, lag_t_v, pair_v, out_v, sem_i, sem_l, sem_t, sem_p):
    wid = lax.axis_index("s") * _NUM_CORES + lax.axis_index("c")

    @pl.when(wid < _ACTIVE)
    def _():
        base = lax.min(wid * _CHUNK, _NUM_ROWS - _CHUNK)
        ci = pltpu.async_copy(idxp_hbm.at[pl.ds(base, _CHUNK)],
                              idxp_v, sem_i)
        cl = pltpu.async_copy(idxl_hbm.at[pl.ds(base, _CHUNK)],
                              idxl_v, sem_l)
        ct = pltpu.async_copy(lag_hbm, lag_t_v, sem_t)
        ci.wait()

        gathers = []
        for g in range(_GROUPS):
            gathers.append(pltpu.async_copy(
                pair_hbm.at[idxp_v[pl.ds(16 * g, 16)]],
                pair_v.at[pl.ds(16 * g, 16)], sem_p))
        cl.wait()
        lidx = [idxl_v[pl.ds(16 * g, 16)] for g in range(_GROUPS)]

        ct.wait()
        lr = [lag_t_v[i, :] for i in range(_NUM_LAGS)]
        d01 = lr[1] - lr[0]
        d23 = lr[3] - lr[2]
        one = jnp.ones((16,), jnp.int32)
        for r in range(_CHUNK):
            bc = _take(lidx[r // 16], jnp.full((16,), r % 16, jnp.int32))
            b0 = (bc & one).astype(jnp.float32)
            b1 = ((bc >> 1) & one).astype(jnp.float32)
            lo = lr[0] + b0 * d01
            hi = lr[2] + b0 * d23
            sel = lo + b1 * (hi - lo)
            out_v[r, pl.ds(_PAIR_DIM, _LAG_DIM)] = sel

        for c in gathers:
            c.wait()
        for r in range(_CHUNK):
            for j in range(_PAIR_DIM // 16):
                out_v[r, pl.ds(j * 16, 16)] = pair_v[r, pl.ds(j * 16, 16)]

        pltpu.sync_copy(out_v, out_hbm.at[pl.ds(base, _CHUNK)])


@jax.jit
def _emb(idxp, idxl, pair_table, lag_table):
    mesh = plsc.VectorSubcoreMesh(core_axis_name="c", subcore_axis_name="s",
                                  num_cores=_NUM_CORES)
    run = functools.partial(
        pl.kernel,
        out_type=jax.ShapeDtypeStruct((_NUM_ROWS, _OUT_DIM), jnp.float32),
        mesh=mesh,
        scratch_types=[
            pltpu.VMEM((_CHUNK,), jnp.int32),
            pltpu.VMEM((_CHUNK,), jnp.int32),
            pltpu.VMEM((_NUM_LAGS, _LAG_DIM), jnp.float32),
            pltpu.VMEM((_CHUNK, _PAIR_DIM), jnp.float32),
            pltpu.VMEM((_CHUNK, _OUT_DIM), jnp.float32),
            pltpu.SemaphoreType.DMA,
            pltpu.SemaphoreType.DMA,
            pltpu.SemaphoreType.DMA,
            pltpu.SemaphoreType.DMA,
        ],
    )(_body)
    return run(idxp, idxl, pair_table, lag_table)


def kernel(target_indices, pair_table, lag_table):
    idx = target_indices.astype(jnp.int32)
    return _emb(idx[:, 0], idx[:, 1], pair_table, lag_table)


# speedup vs baseline: 1.0023x; 1.0023x over previous
import functools

import jax
import jax.numpy as jnp
from jax import lax
from jax.experimental import pallas as pl
from jax.experimental.pallas import tpu as pltpu
from jax.experimental.pallas import tpu_sc as plsc

_NUM_ROWS = 424
_PAIR_DIM = 128
_LAG_DIM = 16
_OUT_DIM = _PAIR_DIM + _LAG_DIM
_NUM_LAGS = 4
_CHUNK = 32
_GROUPS = _CHUNK // 16
_ACTIVE = (_NUM_ROWS + _CHUNK - 1) // _CHUNK
_NUM_CORES = 1


def _take(v, i):
    dnums = lax.GatherDimensionNumbers(
        offset_dims=(), collapsed_slice_dims=(0,), start_index_map=(0,))
    return lax.gather(v, i[:, None], dnums, slice_sizes=(1,),
                      mode=lax.GatherScatterMode.PROMISE_IN_BOUNDS)


def _body(idxp_hbm, idxl_hbm, pair_hbm, lag_hbm, out_hbm,
          idxp_v, idxl_v, lag_t_v, pair_v, out_v,
          sem_i, sem_l, sem_t, sem_p, sem_o):
    wid = lax.axis_index("s") * _NUM_CORES + lax.axis_index("c")

    @pl.when(wid < _ACTIVE)
    def _():
        base = lax.min(wid * _CHUNK, _NUM_ROWS - _CHUNK)
        ci = pltpu.async_copy(idxp_hbm.at[pl.ds(base, _CHUNK)],
                              idxp_v, sem_i)
        cl = pltpu.async_copy(idxl_hbm.at[pl.ds(base, _CHUNK)],
                              idxl_v, sem_l)
        ct = pltpu.async_copy(lag_hbm, lag_t_v, sem_t)
        ci.wait()

        gathers = []
        for g in range(_GROUPS):
            gathers.append(pltpu.async_copy(
                pair_hbm.at[idxp_v[pl.ds(16 * g, 16)]],
                pair_v.at[pl.ds(16 * g, 16)], sem_p))
        cl.wait()
        lidx = [idxl_v[pl.ds(16 * g, 16)] for g in range(_GROUPS)]

        ct.wait()
        lr = [lag_t_v[i, :] for i in range(_NUM_LAGS)]
        d01 = lr[1] - lr[0]
        d23 = lr[3] - lr[2]
        one = jnp.ones((16,), jnp.int32)
        for r in range(_CHUNK):
            bc = _take(lidx[r // 16], jnp.full((16,), r % 16, jnp.int32))
            b0 = (bc & one).astype(jnp.float32)
            b1 = ((bc >> 1) & one).astype(jnp.float32)
            lo = lr[0] + b0 * d01
            hi = lr[2] + b0 * d23
            sel = lo + b1 * (hi - lo)
            out_v[r, pl.ds(_PAIR_DIM, _LAG_DIM)] = sel

        outs = []
        for g in range(_GROUPS):
            gathers[g].wait()
            for r in range(16 * g, 16 * g + 16):
                for j in range(_PAIR_DIM // 16):
                    out_v[r, pl.ds(j * 16, 16)] = pair_v[r, pl.ds(j * 16, 16)]
            outs.append(pltpu.async_copy(
                out_v.at[pl.ds(16 * g, 16)],
                out_hbm.at[pl.ds(base + 16 * g, 16)], sem_o))
        for c in outs:
            c.wait()


@jax.jit
def _emb(idxp, idxl, pair_table, lag_table):
    mesh = plsc.VectorSubcoreMesh(core_axis_name="c", subcore_axis_name="s",
                                  num_cores=_NUM_CORES)
    run = functools.partial(
        pl.kernel,
        out_type=jax.ShapeDtypeStruct((_NUM_ROWS, _OUT_DIM), jnp.float32),
        mesh=mesh,
        scratch_types=[
            pltpu.VMEM((_CHUNK,), jnp.int32),
            pltpu.VMEM((_CHUNK,), jnp.int32),
            pltpu.VMEM((_NUM_LAGS, _LAG_DIM), jnp.float32),
            pltpu.VMEM((_CHUNK, _PAIR_DIM), jnp.float32),
            pltpu.VMEM((_CHUNK, _OUT_DIM), jnp.float32),
            pltpu.SemaphoreType.DMA,
            pltpu.SemaphoreType.DMA,
            pltpu.SemaphoreType.DMA,
            pltpu.SemaphoreType.DMA,
            pltpu.SemaphoreType.DMA,
        ],
    )(_body)
    return run(idxp, idxl, pair_table, lag_table)


def kernel(target_indices, pair_table, lag_table):
    idx = target_indices.astype(jnp.int32)
    return _emb(idx[:, 0], idx[:, 1], pair_table, lag_table)
